# baseline (speedup 1.0000x reference)
import functools

import jax
import jax.numpy as jnp
from jax import lax
from jax.experimental import pallas as pl
from jax.experimental.pallas import tpu as pltpu
from jax.experimental.pallas import tpu_sc as plsc

B = 16384
D = 32
L = 16


@functools.cache
def _build():
    info = plsc.get_sparse_core_info()
    NC, NS = info.num_cores, info.num_subcores
    NW = NC * NS
    BW = B // NW
    NCHUNK = 4
    CW = BW // NCHUNK
    NG = BW // L

    mesh = plsc.VectorSubcoreMesh(core_axis_name="c", subcore_axis_name="s")

    @functools.partial(
        pl.kernel,
        mesh=mesh,
        out_type=jax.ShapeDtypeStruct((B,), jnp.float32),
        scratch_types=[
            pltpu.VMEM((NCHUNK, CW), jnp.int32),
            pltpu.VMEM((NCHUNK, CW), jnp.int32),
            pltpu.VMEM((BW, D), jnp.float32),
            pltpu.VMEM((BW, D), jnp.float32),
            pltpu.VMEM((BW,), jnp.float32),
            pltpu.VMEM((L * (L + 1),), jnp.float32),
            pltpu.SemaphoreType.DMA,
        ],
        compiler_params=pltpu.CompilerParams(
            needs_layout_passes=False, use_tc_tiling_on_sc=False),
    )
    def scores_kernel(uid_hbm, iid_hbm, uemb_hbm, iemb_hbm, out_hbm,
                      uidx_v, iidx_v, urows_v, vrows_v, scores_v, pad_v, sem):
        wid = lax.axis_index("s") * NC + lax.axis_index("c")
        base = wid * BW

        copies = []
        for c in range(NCHUNK):
            pltpu.sync_copy(uid_hbm.at[pl.ds(base + c * CW, CW)], uidx_v.at[c])
            pltpu.sync_copy(iid_hbm.at[pl.ds(base + c * CW, CW)], iidx_v.at[c])
            copies.append(pltpu.async_copy(
                uemb_hbm.at[uidx_v.at[c]], urows_v.at[pl.ds(c * CW, CW)], sem))
            copies.append(pltpu.async_copy(
                iemb_hbm.at[iidx_v.at[c]], vrows_v.at[pl.ds(c * CW, CW)], sem))
        for cp in copies:
            cp.wait()

        lanes = lax.iota(jnp.int32, L)

        def group(g, carry):
            for k in range(L):
                row = g * L + k
                s = (urows_v[row, pl.ds(0, L)] * vrows_v[row, pl.ds(0, L)]
                     + urows_v[row, pl.ds(L, L)] * vrows_v[row, pl.ds(L, L)])
                pad_v[pl.ds(k * (L + 1), L)] = s
            acc = jnp.zeros((L,), jnp.float32)
            for j in range(L):
                col = plsc.load_gather(pad_v, [lanes * (L + 1) + j])
                acc = acc + col
            scores_v[pl.ds(g * L, L)] = acc
            return carry

        lax.fori_loop(0, NG, group, 0)

        pltpu.sync_copy(scores_v, out_hbm.at[pl.ds(base, BW)])

    return scores_kernel


def kernel(user_ids, item_ids, user_emb, item_emb):
    return _build()(user_ids.astype(jnp.int32), item_ids.astype(jnp.int32),
                    user_emb, item_emb)


# speedup vs baseline: 1.0453x; 1.0453x over previous
import functools

import jax
import jax.numpy as jnp
from jax import lax
from jax.experimental import pallas as pl
from jax.experimental.pallas import tpu as pltpu
from jax.experimental.pallas import tpu_sc as plsc

N = 1000000
B = 16384
D = 32
L = 16

_IDS_PER_TC_BLK = 2048
_TC_GRID = (N + _IDS_PER_TC_BLK - 1) // _IDS_PER_TC_BLK
_OUT_ROWS_PER_BLK = _IDS_PER_TC_BLK * D // 128
_R = _TC_GRID * _OUT_ROWS_PER_BLK


def _pack_block(inT_ref, out_ref):
    x = inT_ref[...]
    for q in range(_IDS_PER_TC_BLK // 512):
        parts = [x[:, 128 * (4 * q + c):128 * (4 * q + c) + 128].T
                 for c in range(4)]
        out_ref[pl.ds(128 * q, 128), :] = jnp.concatenate(parts, axis=1)


@jax.jit
def _pack_rows(table):
    tT = table.T
    return pl.pallas_call(
        _pack_block,
        grid=(_TC_GRID,),
        in_specs=[pl.BlockSpec((D, _IDS_PER_TC_BLK), lambda j: (0, j))],
        out_specs=pl.BlockSpec((_OUT_ROWS_PER_BLK, 128), lambda j: (j, 0)),
        out_shape=jax.ShapeDtypeStruct((_R, 128), jnp.float32),
    )(tT)


@functools.cache
def _build_sc():
    info = plsc.get_sparse_core_info()
    NC, NS = info.num_cores, info.num_subcores
    NW = NC * NS
    BW = B // NW
    NCHUNK = 4
    CW = BW // NCHUNK
    NG = CW // L

    mesh = plsc.VectorSubcoreMesh(core_axis_name="c", subcore_axis_name="s")

    @functools.partial(
        pl.kernel,
        mesh=mesh,
        out_type=jax.ShapeDtypeStruct((B,), jnp.float32),
        scratch_types=[
            pltpu.VMEM((NCHUNK, CW), jnp.int32),
            pltpu.VMEM((NCHUNK, CW), jnp.int32),
            pltpu.VMEM((NCHUNK, CW), jnp.int32),
            pltpu.VMEM((NCHUNK, CW), jnp.int32),
            pltpu.VMEM((CW, 128), jnp.float32),
            pltpu.VMEM((CW, 128), jnp.float32),
            pltpu.VMEM((BW,), jnp.float32),
            pltpu.VMEM((L * (L + 1),), jnp.float32),
            pltpu.SemaphoreType.DMA,
        ],
        compiler_params=pltpu.CompilerParams(
            needs_layout_passes=False, use_tc_tiling_on_sc=False),
    )
    def scores_kernel(uid_hbm, iid_hbm, upk_hbm, ipk_hbm, out_hbm,
                      uids_v, iids_v, uq_v, iq_v, urows_v, vrows_v,
                      scores_v, pad_v, sem):
        wid = lax.axis_index("s") * NC + lax.axis_index("c")
        base = wid * BW

        for c in range(NCHUNK):
            pltpu.sync_copy(uid_hbm.at[pl.ds(base + c * CW, CW)], uids_v.at[c])
            pltpu.sync_copy(iid_hbm.at[pl.ds(base + c * CW, CW)], iids_v.at[c])
            for g in range(NG):
                sl = pl.ds(g * L, L)
                for ids_v, q_v in ((uids_v, uq_v), (iids_v, iq_v)):
                    r = ids_v[c, sl]
                    q = ((r >> 9) << 7) | (r & 127)
                    q_v[c, sl] = q

        lanes = lax.iota(jnp.int32, L)

        for c in range(NCHUNK):
            cu = pltpu.async_copy(upk_hbm.at[uq_v.at[c]], urows_v, sem)
            cv = pltpu.async_copy(ipk_hbm.at[iq_v.at[c]], vrows_v, sem)
            cu.wait()
            cv.wait()

            for g in range(NG):
                sl = pl.ds(g * L, L)
                uoff = ((uids_v[c, sl] >> 7) & 3) << 5
                ioff = ((iids_v[c, sl] >> 7) & 3) << 5
                for k in range(L):
                    row = g * L + k
                    uo = uoff[k]
                    io = ioff[k]
                    s = (urows_v[row, pl.ds(uo, L)]
                         * vrows_v[row, pl.ds(io, L)]
                         + urows_v[row, pl.ds(uo + L, L)]
                         * vrows_v[row, pl.ds(io + L, L)])
                    pad_v[pl.ds(k * (L + 1), L)] = s
                acc = jnp.zeros((L,), jnp.float32)
                for j in range(L):
                    col = plsc.load_gather(pad_v, [lanes * (L + 1) + j])
                    acc = acc + col
                scores_v[pl.ds(c * CW + g * L, L)] = acc

        pltpu.sync_copy(scores_v, out_hbm.at[pl.ds(base, BW)])

    return scores_kernel


def kernel(user_ids, item_ids, user_emb, item_emb):
    u_pk = _pack_rows(user_emb)
    i_pk = _pack_rows(item_emb)
    return _build_sc()(user_ids.astype(jnp.int32), item_ids.astype(jnp.int32),
                       u_pk, i_pk)


# speedup vs baseline: 3.7838x; 3.6200x over previous
import functools

import jax
import jax.numpy as jnp
from jax import lax
from jax.experimental import pallas as pl
from jax.experimental.pallas import tpu as pltpu
from jax.experimental.pallas import tpu_sc as plsc

N = 1000000
B = 16384
D = 32
L = 16

_IDS_PER_TC_BLK = 16384
_TC_GRID = (N + _IDS_PER_TC_BLK - 1) // _IDS_PER_TC_BLK
_OUT_ROWS_PER_BLK = _IDS_PER_TC_BLK * D // 128
_R = _TC_GRID * _OUT_ROWS_PER_BLK


def _pack_block(inT_ref, out_ref):
    x = inT_ref[...]
    nq = _IDS_PER_TC_BLK // 512
    bands = [jnp.concatenate(
        [x[:, 512 * q + 128 * c:512 * q + 128 * c + 128] for q in range(nq)],
        axis=1) for c in range(4)]
    xb = jnp.concatenate(bands, axis=0)
    eye = jnp.where(lax.broadcasted_iota(jnp.int32, (128, 128), 0)
                    == lax.broadcasted_iota(jnp.int32, (128, 128), 1),
                    1.0, 0.0)
    out_ref[...] = lax.dot_general(xb, eye, (((0,), (0,)), ((), ())),
                                   preferred_element_type=jnp.float32)


@jax.jit
def _pack_rows(table):
    tT = table.T
    return pl.pallas_call(
        _pack_block,
        grid=(_TC_GRID,),
        in_specs=[pl.BlockSpec((D, _IDS_PER_TC_BLK), lambda j: (0, j))],
        out_specs=pl.BlockSpec((_OUT_ROWS_PER_BLK, 128), lambda j: (j, 0)),
        out_shape=jax.ShapeDtypeStruct((_R, 128), jnp.float32),
    )(tT)


@functools.cache
def _build_sc():
    info = plsc.get_sparse_core_info()
    NC, NS = info.num_cores, info.num_subcores
    NW = NC * NS
    BW = B // NW
    NCHUNK = 4
    CW = BW // NCHUNK
    NG = CW // L

    mesh = plsc.VectorSubcoreMesh(core_axis_name="c", subcore_axis_name="s")

    @functools.partial(
        pl.kernel,
        mesh=mesh,
        out_type=jax.ShapeDtypeStruct((B,), jnp.float32),
        scratch_types=[
            pltpu.VMEM((NCHUNK, CW), jnp.int32),
            pltpu.VMEM((NCHUNK, CW), jnp.int32),
            pltpu.VMEM((NCHUNK, CW), jnp.int32),
            pltpu.VMEM((NCHUNK, CW), jnp.int32),
            pltpu.VMEM((CW, 128), jnp.float32),
            pltpu.VMEM((CW, 128), jnp.float32),
            pltpu.VMEM((BW,), jnp.float32),
            pltpu.VMEM((L * (L + 1),), jnp.float32),
            pltpu.SemaphoreType.DMA,
        ],
        compiler_params=pltpu.CompilerParams(
            needs_layout_passes=False, use_tc_tiling_on_sc=False),
    )
    def scores_kernel(uid_hbm, iid_hbm, upk_hbm, ipk_hbm, out_hbm,
                      uids_v, iids_v, uq_v, iq_v, urows_v, vrows_v,
                      scores_v, pad_v, sem):
        wid = lax.axis_index("s") * NC + lax.axis_index("c")
        base = wid * BW

        for c in range(NCHUNK):
            pltpu.sync_copy(uid_hbm.at[pl.ds(base + c * CW, CW)], uids_v.at[c])
            pltpu.sync_copy(iid_hbm.at[pl.ds(base + c * CW, CW)], iids_v.at[c])
            for g in range(NG):
                sl = pl.ds(g * L, L)
                for ids_v, q_v in ((uids_v, uq_v), (iids_v, iq_v)):
                    r = ids_v[c, sl]
                    q = ((r >> 9) << 7) | (r & 127)
                    q_v[c, sl] = q

        lanes = lax.iota(jnp.int32, L)

        for c in range(NCHUNK):
            cu = pltpu.async_copy(upk_hbm.at[uq_v.at[c]], urows_v, sem)
            cv = pltpu.async_copy(ipk_hbm.at[iq_v.at[c]], vrows_v, sem)
            cu.wait()
            cv.wait()

            for g in range(NG):
                sl = pl.ds(g * L, L)
                uoff = ((uids_v[c, sl] >> 7) & 3) << 5
                ioff = ((iids_v[c, sl] >> 7) & 3) << 5
                for k in range(L):
                    row = g * L + k
                    uo = uoff[k]
                    io = ioff[k]
                    s = (urows_v[row, pl.ds(uo, L)]
                         * vrows_v[row, pl.ds(io, L)]
                         + urows_v[row, pl.ds(uo + L, L)]
                         * vrows_v[row, pl.ds(io + L, L)])
                    pad_v[pl.ds(k * (L + 1), L)] = s
                acc = jnp.zeros((L,), jnp.float32)
                for j in range(L):
                    col = plsc.load_gather(pad_v, [lanes * (L + 1) + j])
                    acc = acc + col
                scores_v[pl.ds(c * CW + g * L, L)] = acc

        pltpu.sync_copy(scores_v, out_hbm.at[pl.ds(base, BW)])

    return scores_kernel


def kernel(user_ids, item_ids, user_emb, item_emb):
    u_pk = _pack_rows(user_emb)
    i_pk = _pack_rows(item_emb)
    return _build_sc()(user_ids.astype(jnp.int32), item_ids.astype(jnp.int32),
                       u_pk, i_pk)


# speedup vs baseline: 4.3257x; 1.1432x over previous
import functools

import jax
import jax.numpy as jnp
from jax import lax
from jax.experimental import pallas as pl
from jax.experimental.pallas import tpu as pltpu
from jax.experimental.pallas import tpu_sc as plsc

N = 1000000
B = 16384
D = 32
L = 16

_IDS_PER_TC_BLK = 32768
_TC_GRID = (N + _IDS_PER_TC_BLK - 1) // _IDS_PER_TC_BLK
_OUT_ROWS_PER_BLK = _IDS_PER_TC_BLK * D // 128
_R = _TC_GRID * _OUT_ROWS_PER_BLK


def _pack_block(inT_ref, out_ref):
    x = inT_ref[...]
    nq = _IDS_PER_TC_BLK // 512
    bands = [jnp.concatenate(
        [x[:, 512 * q + 128 * c:512 * q + 128 * c + 128] for q in range(nq)],
        axis=1) for c in range(4)]
    xb = jnp.concatenate(bands, axis=0)
    eye = jnp.where(lax.broadcasted_iota(jnp.int32, (128, 128), 0)
                    == lax.broadcasted_iota(jnp.int32, (128, 128), 1),
                    1.0, 0.0)
    out_ref[...] = lax.dot_general(xb, eye, (((0,), (0,)), ((), ())),
                                   preferred_element_type=jnp.float32)


@jax.jit
def _pack_rows(table):
    tT = table.T
    return pl.pallas_call(
        _pack_block,
        grid=(_TC_GRID,),
        in_specs=[pl.BlockSpec((D, _IDS_PER_TC_BLK), lambda j: (0, j))],
        out_specs=pl.BlockSpec((_OUT_ROWS_PER_BLK, 128), lambda j: (j, 0)),
        out_shape=jax.ShapeDtypeStruct((_R, 128), jnp.float32),
    )(tT)


@functools.cache
def _build_sc():
    info = plsc.get_sparse_core_info()
    NC, NS = info.num_cores, info.num_subcores
    NW = NC * NS
    BW = B // NW
    NCHUNK = 4
    CW = BW // NCHUNK
    NG = CW // L

    mesh = plsc.VectorSubcoreMesh(core_axis_name="c", subcore_axis_name="s")

    @functools.partial(
        pl.kernel,
        mesh=mesh,
        out_type=jax.ShapeDtypeStruct((B,), jnp.float32),
        scratch_types=[
            pltpu.VMEM((NCHUNK, CW), jnp.int32),
            pltpu.VMEM((NCHUNK, CW), jnp.int32),
            pltpu.VMEM((NCHUNK, CW), jnp.int32),
            pltpu.VMEM((NCHUNK, CW), jnp.int32),
            pltpu.VMEM((CW, 128), jnp.float32),
            pltpu.VMEM((CW, 128), jnp.float32),
            pltpu.VMEM((BW,), jnp.float32),
            pltpu.VMEM((L * (L + 1),), jnp.float32),
            pltpu.SemaphoreType.DMA,
        ],
        compiler_params=pltpu.CompilerParams(
            needs_layout_passes=False, use_tc_tiling_on_sc=False),
    )
    def scores_kernel(uid_hbm, iid_hbm, upk_hbm, ipk_hbm, out_hbm,
                      uids_v, iids_v, uq_v, iq_v, urows_v, vrows_v,
                      scores_v, pad_v, sem):
        wid = lax.axis_index("s") * NC + lax.axis_index("c")
        base = wid * BW

        for c in range(NCHUNK):
            pltpu.sync_copy(uid_hbm.at[pl.ds(base + c * CW, CW)], uids_v.at[c])
            pltpu.sync_copy(iid_hbm.at[pl.ds(base + c * CW, CW)], iids_v.at[c])
            for g in range(NG):
                sl = pl.ds(g * L, L)
                for ids_v, q_v in ((uids_v, uq_v), (iids_v, iq_v)):
                    r = ids_v[c, sl]
                    q = ((r >> 9) << 7) | (r & 127)
                    q_v[c, sl] = q

        lanes = lax.iota(jnp.int32, L)

        for c in range(NCHUNK):
            cu = pltpu.async_copy(upk_hbm.at[uq_v.at[c]], urows_v, sem)
            cv = pltpu.async_copy(ipk_hbm.at[iq_v.at[c]], vrows_v, sem)
            cu.wait()
            cv.wait()

            for g in range(NG):
                sl = pl.ds(g * L, L)
                uoff = ((uids_v[c, sl] >> 7) & 3) << 5
                ioff = ((iids_v[c, sl] >> 7) & 3) << 5
                for k in range(L):
                    row = g * L + k
                    uo = uoff[k]
                    io = ioff[k]
                    s = (urows_v[row, pl.ds(uo, L)]
                         * vrows_v[row, pl.ds(io, L)]
                         + urows_v[row, pl.ds(uo + L, L)]
                         * vrows_v[row, pl.ds(io + L, L)])
                    pad_v[pl.ds(k * (L + 1), L)] = s
                acc = jnp.zeros((L,), jnp.float32)
                for j in range(L):
                    col = plsc.load_gather(pad_v, [lanes * (L + 1) + j])
                    acc = acc + col
                scores_v[pl.ds(c * CW + g * L, L)] = acc

        pltpu.sync_copy(scores_v, out_hbm.at[pl.ds(base, BW)])

    return scores_kernel


def kernel(user_ids, item_ids, user_emb, item_emb):
    u_pk = _pack_rows(user_emb)
    i_pk = _pack_rows(item_emb)
    return _build_sc()(user_ids.astype(jnp.int32), item_ids.astype(jnp.int32),
                       u_pk, i_pk)


# speedup vs baseline: 4.3592x; 1.0077x over previous
import functools

import jax
import jax.numpy as jnp
from jax import lax
from jax.experimental import pallas as pl
from jax.experimental.pallas import tpu as pltpu
from jax.experimental.pallas import tpu_sc as plsc

N = 1000000
B = 16384
D = 32
L = 16

_IDS_PER_TC_BLK = 65536
_TC_GRID = (N + _IDS_PER_TC_BLK - 1) // _IDS_PER_TC_BLK
_OUT_ROWS_PER_BLK = _IDS_PER_TC_BLK * D // 128
_R = _TC_GRID * _OUT_ROWS_PER_BLK


def _pack_block(inT_ref, out_ref):
    x = inT_ref[...]
    nq = _IDS_PER_TC_BLK // 512
    bands = [jnp.concatenate(
        [x[:, 512 * q + 128 * c:512 * q + 128 * c + 128] for q in range(nq)],
        axis=1) for c in range(4)]
    xb = jnp.concatenate(bands, axis=0)
    eye = jnp.where(lax.broadcasted_iota(jnp.int32, (128, 128), 0)
                    == lax.broadcasted_iota(jnp.int32, (128, 128), 1),
                    1.0, 0.0)
    out_ref[...] = lax.dot_general(xb, eye, (((0,), (0,)), ((), ())),
                                   preferred_element_type=jnp.float32)


@jax.jit
def _pack_rows(table):
    tT = table.T
    return pl.pallas_call(
        _pack_block,
        grid=(_TC_GRID,),
        in_specs=[pl.BlockSpec((D, _IDS_PER_TC_BLK), lambda j: (0, j))],
        out_specs=pl.BlockSpec((_OUT_ROWS_PER_BLK, 128), lambda j: (j, 0)),
        out_shape=jax.ShapeDtypeStruct((_R, 128), jnp.float32),
    )(tT)


@functools.cache
def _build_sc():
    info = plsc.get_sparse_core_info()
    NC, NS = info.num_cores, info.num_subcores
    NW = NC * NS
    BW = B // NW
    NCHUNK = 4
    CW = BW // NCHUNK
    NG = CW // L

    mesh = plsc.VectorSubcoreMesh(core_axis_name="c", subcore_axis_name="s")

    @functools.partial(
        pl.kernel,
        mesh=mesh,
        out_type=jax.ShapeDtypeStruct((B,), jnp.float32),
        scratch_types=[
            pltpu.VMEM((NCHUNK, CW), jnp.int32),
            pltpu.VMEM((NCHUNK, CW), jnp.int32),
            pltpu.VMEM((NCHUNK, CW), jnp.int32),
            pltpu.VMEM((NCHUNK, CW), jnp.int32),
            pltpu.VMEM((CW, 128), jnp.float32),
            pltpu.VMEM((CW, 128), jnp.float32),
            pltpu.VMEM((BW,), jnp.float32),
            pltpu.VMEM((L * (L + 1),), jnp.float32),
            pltpu.SemaphoreType.DMA,
        ],
        compiler_params=pltpu.CompilerParams(
            needs_layout_passes=False, use_tc_tiling_on_sc=False),
    )
    def scores_kernel(uid_hbm, iid_hbm, upk_hbm, ipk_hbm, out_hbm,
                      uids_v, iids_v, uq_v, iq_v, urows_v, vrows_v,
                      scores_v, pad_v, sem):
        wid = lax.axis_index("s") * NC + lax.axis_index("c")
        base = wid * BW

        for c in range(NCHUNK):
            pltpu.sync_copy(uid_hbm.at[pl.ds(base + c * CW, CW)], uids_v.at[c])
            pltpu.sync_copy(iid_hbm.at[pl.ds(base + c * CW, CW)], iids_v.at[c])
            for g in range(NG):
                sl = pl.ds(g * L, L)
                for ids_v, q_v in ((uids_v, uq_v), (iids_v, iq_v)):
                    r = ids_v[c, sl]
                    q = ((r >> 9) << 7) | (r & 127)
                    q_v[c, sl] = q

        lanes = lax.iota(jnp.int32, L)

        for c in range(NCHUNK):
            cu = pltpu.async_copy(upk_hbm.at[uq_v.at[c]], urows_v, sem)
            cv = pltpu.async_copy(ipk_hbm.at[iq_v.at[c]], vrows_v, sem)
            cu.wait()
            cv.wait()

            for g in range(NG):
                sl = pl.ds(g * L, L)
                uoff = ((uids_v[c, sl] >> 7) & 3) << 5
                ioff = ((iids_v[c, sl] >> 7) & 3) << 5
                for k in range(L):
                    row = g * L + k
                    uo = uoff[k]
                    io = ioff[k]
                    s = (urows_v[row, pl.ds(uo, L)]
                         * vrows_v[row, pl.ds(io, L)]
                         + urows_v[row, pl.ds(uo + L, L)]
                         * vrows_v[row, pl.ds(io + L, L)])
                    pad_v[pl.ds(k * (L + 1), L)] = s
                acc = jnp.zeros((L,), jnp.float32)
                for j in range(L):
                    col = plsc.load_gather(pad_v, [lanes * (L + 1) + j])
                    acc = acc + col
                scores_v[pl.ds(c * CW + g * L, L)] = acc

        pltpu.sync_copy(scores_v, out_hbm.at[pl.ds(base, BW)])

    return scores_kernel


def kernel(user_ids, item_ids, user_emb, item_emb):
    u_pk = _pack_rows(user_emb)
    i_pk = _pack_rows(item_emb)
    return _build_sc()(user_ids.astype(jnp.int32), item_ids.astype(jnp.int32),
                       u_pk, i_pk)


# speedup vs baseline: 4.4393x; 1.0184x over previous
import functools

import jax
import jax.numpy as jnp
from jax import lax
from jax.experimental import pallas as pl
from jax.experimental.pallas import tpu as pltpu
from jax.experimental.pallas import tpu_sc as plsc

N = 1000000
B = 16384
D = 32
L = 16

_IDS_PER_TC_BLK = 65536
_TC_GRID = (N + _IDS_PER_TC_BLK - 1) // _IDS_PER_TC_BLK
_OUT_ROWS_PER_BLK = _IDS_PER_TC_BLK * D // 128
_R = _TC_GRID * _OUT_ROWS_PER_BLK


def _pack_block(inT_ref, out_ref):
    x = inT_ref[...]
    nq = _IDS_PER_TC_BLK // 512
    bands = [jnp.concatenate(
        [x[:, 512 * q + 128 * c:512 * q + 128 * c + 128] for q in range(nq)],
        axis=1) for c in range(4)]
    xb = jnp.concatenate(bands, axis=0)
    eye = jnp.where(lax.broadcasted_iota(jnp.int32, (128, 128), 0)
                    == lax.broadcasted_iota(jnp.int32, (128, 128), 1),
                    1.0, 0.0)
    out_ref[...] = lax.dot_general(xb, eye, (((0,), (0,)), ((), ())),
                                   preferred_element_type=jnp.float32)


@jax.jit
def _pack_rows(table):
    tT = table.T
    return pl.pallas_call(
        _pack_block,
        grid=(_TC_GRID,),
        in_specs=[pl.BlockSpec((D, _IDS_PER_TC_BLK), lambda j: (0, j))],
        out_specs=pl.BlockSpec((_OUT_ROWS_PER_BLK, 128), lambda j: (j, 0)),
        out_shape=jax.ShapeDtypeStruct((_R, 128), jnp.float32),
    )(tT)


@functools.cache
def _build_sc():
    info = plsc.get_sparse_core_info()
    NC, NS = info.num_cores, info.num_subcores
    NW = NC * NS
    BW = B // NW
    NCHUNK = 4
    CW = BW // NCHUNK
    NG = CW // L

    mesh = plsc.VectorSubcoreMesh(core_axis_name="c", subcore_axis_name="s")

    @functools.partial(
        pl.kernel,
        mesh=mesh,
        out_type=jax.ShapeDtypeStruct((B,), jnp.float32),
        scratch_types=[
            pltpu.VMEM((NCHUNK, CW), jnp.int32),
            pltpu.VMEM((NCHUNK, CW), jnp.int32),
            pltpu.VMEM((NCHUNK, CW), jnp.int32),
            pltpu.VMEM((NCHUNK, CW), jnp.int32),
            pltpu.VMEM((2, CW, 128), jnp.float32),
            pltpu.VMEM((2, CW, 128), jnp.float32),
            pltpu.VMEM((BW,), jnp.float32),
            pltpu.VMEM((L * (L + 1),), jnp.float32),
            pltpu.SemaphoreType.DMA,
            pltpu.SemaphoreType.DMA,
        ],
        compiler_params=pltpu.CompilerParams(
            needs_layout_passes=False, use_tc_tiling_on_sc=False),
    )
    def scores_kernel(uid_hbm, iid_hbm, upk_hbm, ipk_hbm, out_hbm,
                      uids_v, iids_v, uq_v, iq_v, urows_v, vrows_v,
                      scores_v, pad_v, sem0, sem1):
        wid = lax.axis_index("s") * NC + lax.axis_index("c")
        base = wid * BW

        for c in range(NCHUNK):
            pltpu.sync_copy(uid_hbm.at[pl.ds(base + c * CW, CW)], uids_v.at[c])
            pltpu.sync_copy(iid_hbm.at[pl.ds(base + c * CW, CW)], iids_v.at[c])
            for g in range(NG):
                sl = pl.ds(g * L, L)
                for ids_v, q_v in ((uids_v, uq_v), (iids_v, iq_v)):
                    r = ids_v[c, sl]
                    q = ((r >> 9) << 7) | (r & 127)
                    q_v[c, sl] = q

        lanes = lax.iota(jnp.int32, L)
        sems = (sem0, sem1)
        pend = {}

        def issue(c):
            buf = c & 1
            pend[c] = (
                pltpu.async_copy(upk_hbm.at[uq_v.at[c]], urows_v.at[buf],
                                 sems[buf]),
                pltpu.async_copy(ipk_hbm.at[iq_v.at[c]], vrows_v.at[buf],
                                 sems[buf]))

        issue(0)
        issue(1)
        for c in range(NCHUNK):
            buf = c & 1
            for cp in pend.pop(c):
                cp.wait()

            for g in range(NG):
                sl = pl.ds(g * L, L)
                uoff = ((uids_v[c, sl] >> 7) & 3) << 5
                ioff = ((iids_v[c, sl] >> 7) & 3) << 5
                for k in range(L):
                    row = g * L + k
                    uo = uoff[k]
                    io = ioff[k]
                    s = (urows_v[buf, row, pl.ds(uo, L)]
                         * vrows_v[buf, row, pl.ds(io, L)]
                         + urows_v[buf, row, pl.ds(uo + L, L)]
                         * vrows_v[buf, row, pl.ds(io + L, L)])
                    pad_v[pl.ds(k * (L + 1), L)] = s
                acc = jnp.zeros((L,), jnp.float32)
                for j in range(L):
                    col = plsc.load_gather(pad_v, [lanes * (L + 1) + j])
                    acc = acc + col
                scores_v[pl.ds(c * CW + g * L, L)] = acc

            if c + 2 < NCHUNK:
                issue(c + 2)

        pltpu.sync_copy(scores_v, out_hbm.at[pl.ds(base, BW)])

    return scores_kernel


def kernel(user_ids, item_ids, user_emb, item_emb):
    u_pk = _pack_rows(user_emb)
    i_pk = _pack_rows(item_emb)
    return _build_sc()(user_ids.astype(jnp.int32), item_ids.astype(jnp.int32),
                       u_pk, i_pk)
